# baseline (speedup 1.0000x reference)
import functools

import jax
import jax.numpy as jnp
from jax import lax
from jax.experimental import pallas as pl
from jax.experimental.pallas import tpu as pltpu
from jax.experimental.pallas import tpu_sc as plsc

EMBED_DIM = 64
NUM_CORES = 2
NUM_SUBCORES = 16
NUM_WORKERS = NUM_CORES * NUM_SUBCORES
CW = 256
NBUF = 4
LOOKAHEAD = 3


@functools.partial(jax.jit, static_argnums=(2,))
def _sc_gather(idx_flat, table, n):
  per_w = n // NUM_WORKERS
  nchunk = per_w // CW
  mesh = plsc.VectorSubcoreMesh(
      core_axis_name="c", subcore_axis_name="s",
      num_cores=NUM_CORES, num_subcores=NUM_SUBCORES)

  @functools.partial(
      pl.kernel,
      out_type=jax.ShapeDtypeStruct((n, EMBED_DIM), jnp.float32),
      mesh=mesh,
      compiler_params=pltpu.CompilerParams(use_tc_tiling_on_sc=False),
      scratch_types=[
          pltpu.VMEM((per_w,), jnp.int32),
          pltpu.VMEM((NBUF, CW, EMBED_DIM), jnp.float32),
          pltpu.SemaphoreType.DMA((NBUF,)),
          pltpu.SemaphoreType.DMA((NBUF,)),
      ],
  )
  def k(idx_hbm, table_hbm, out_hbm, idx_v, rows_v, gsem, ssem):
    wid = lax.axis_index("s") * NUM_CORES + lax.axis_index("c")
    row0 = wid * per_w
    pltpu.sync_copy(idx_hbm.at[pl.ds(row0, per_w)], idx_v)

    def out_slab(j):
      return out_hbm.at[pl.ds(row0 + j * CW, CW)]

    def fire_gather(j, slot):
      off = pl.multiple_of(j * CW, CW)
      pltpu.async_copy(table_hbm.at[idx_v.at[pl.ds(off, CW)]],
                       rows_v.at[slot], gsem.at[slot])

    for b in range(LOOKAHEAD):
      fire_gather(jnp.int32(b), b)

    ngroups = nchunk // NBUF

    def group(g, _):
      j0 = g * NBUF
      for u in range(NBUF):
        j = j0 + u
        pltpu.make_async_copy(table_hbm.at[idx_v.at[pl.ds(0, CW)]],
                              rows_v.at[u], gsem.at[u]).wait()
        pltpu.async_copy(rows_v.at[u], out_slab(j), ssem.at[u])
        nslot = (u + LOOKAHEAD) % NBUF
        jn = j + LOOKAHEAD
        jprev = jn - NBUF

        @pl.when(jn < nchunk)
        def _():
          @pl.when(jprev >= 0)
          def _():
            pltpu.make_async_copy(rows_v.at[nslot], out_slab(jprev),
                                  ssem.at[nslot]).wait()
          fire_gather(jn, nslot)
      return 0

    lax.fori_loop(0, ngroups, group, 0)

    for d in range(NBUF):
      j = nchunk - NBUF + d
      pltpu.make_async_copy(rows_v.at[j % NBUF], out_slab(jnp.int32(j)),
                            ssem.at[j % NBUF]).wait()

  return k(idx_flat, table)


def kernel(indices, feature_tensor):
  bsz, hlen = indices.shape
  idx_flat = indices.astype(jnp.int32).reshape(-1)
  out2 = _sc_gather(idx_flat, feature_tensor, bsz * hlen)
  return out2.reshape(bsz, hlen, EMBED_DIM)


# speedup vs baseline: 1.3275x; 1.3275x over previous
import functools

import jax
import jax.numpy as jnp
from jax import lax
from jax.experimental import pallas as pl
from jax.experimental.pallas import tpu as pltpu
from jax.experimental.pallas import tpu_sc as plsc

EMBED_DIM = 64
NUM_CORES = 2
NUM_SUBCORES = 16
NUM_WORKERS = NUM_CORES * NUM_SUBCORES
CW = 256
NBUF = 4
LOOKAHEAD = 3


@functools.partial(jax.jit, static_argnums=(2,))
def _sc_gather(idx_flat, table, n):
  per_w = n // NUM_WORKERS
  nchunk = per_w // CW
  mesh = plsc.VectorSubcoreMesh(
      core_axis_name="c", subcore_axis_name="s",
      num_cores=NUM_CORES, num_subcores=NUM_SUBCORES)

  @functools.partial(
      pl.kernel,
      out_type=jax.ShapeDtypeStruct((n, 2 * EMBED_DIM), jnp.float32),
      mesh=mesh,
      compiler_params=pltpu.CompilerParams(use_tc_tiling_on_sc=False),
      scratch_types=[
          pltpu.VMEM((per_w,), jnp.int32),
          pltpu.VMEM((NBUF, CW, EMBED_DIM), jnp.float32),
          pltpu.SemaphoreType.DMA((NBUF,)),
          pltpu.SemaphoreType.DMA((NBUF,)),
      ],
  )
  def k(idx_hbm, table_hbm, out_hbm, idx_v, rows_v, gsem, ssem):
    wid = lax.axis_index("s") * NUM_CORES + lax.axis_index("c")
    row0 = wid * per_w
    pltpu.sync_copy(idx_hbm.at[pl.ds(row0, per_w)], idx_v)

    def out_slab(j):
      return out_hbm.at[pl.ds(row0 + j * CW, CW), pl.ds(0, EMBED_DIM)]

    def fire_gather(j, slot):
      off = pl.multiple_of(j * CW, CW)
      pltpu.async_copy(table_hbm.at[idx_v.at[pl.ds(off, CW)]],
                       rows_v.at[slot], gsem.at[slot])

    for b in range(LOOKAHEAD):
      fire_gather(jnp.int32(b), b)

    ngroups = nchunk // NBUF

    def group(g, _):
      j0 = g * NBUF
      for u in range(NBUF):
        j = j0 + u
        pltpu.make_async_copy(table_hbm.at[idx_v.at[pl.ds(0, CW)]],
                              rows_v.at[u], gsem.at[u]).wait()
        pltpu.async_copy(rows_v.at[u], out_slab(j), ssem.at[u])
        nslot = (u + LOOKAHEAD) % NBUF
        jn = j + LOOKAHEAD
        jprev = jn - NBUF

        @pl.when(jn < nchunk)
        def _():
          @pl.when(jprev >= 0)
          def _():
            pltpu.make_async_copy(rows_v.at[nslot], out_slab(jprev),
                                  ssem.at[nslot]).wait()
          fire_gather(jn, nslot)
      return 0

    lax.fori_loop(0, ngroups, group, 0)

    for d in range(NBUF):
      j = nchunk - NBUF + d
      pltpu.make_async_copy(rows_v.at[j % NBUF], out_slab(jnp.int32(j)),
                            ssem.at[j % NBUF]).wait()

  return k(idx_flat, table)


def kernel(indices, feature_tensor):
  bsz, hlen = indices.shape
  idx_flat = indices.astype(jnp.int32).reshape(-1)
  out2 = _sc_gather(idx_flat, feature_tensor, bsz * hlen)
  return out2[:, :EMBED_DIM].reshape(bsz, hlen, EMBED_DIM)
